# Initial kernel scaffold; baseline (speedup 1.0000x reference)
import jax
import jax.numpy as jnp
from jax import lax
from jax.experimental import pallas as pl
from jax.experimental.pallas import tpu as pltpu

_NEG_POS_RATIO = 3


def _ordered_u32(x):
    b = lax.bitcast_convert_type(x, jnp.int32)
    key = b ^ ((b >> 31) | jnp.int32(-2147483648))
    return lax.bitcast_convert_type(key, jnp.uint32)


def _stage_a_kernel(conf_ref, lab_ref, pred_ref, gt_ref,
                    ukey_ref, ce_ref, sl1_ref):
    c = conf_ref[0]
    lab2d = lab_ref[...]
    P, C = c.shape
    m = jnp.max(c, axis=-1, keepdims=True)
    lse = jnp.log(jnp.sum(jnp.exp(c - m), axis=-1, keepdims=True)) + m
    bg = lse[:, 0:1] - c[:, 0:1]
    lab_col = lab2d.reshape(P, 1)
    cls_iota = lax.broadcasted_iota(jnp.int32, (P, C), 1)
    picked = jnp.sum(jnp.where(cls_iota == lab_col, c, 0.0),
                     axis=-1, keepdims=True)
    ce = lse - picked
    ukey_ref[...] = _ordered_u32(bg).reshape(1, P)
    ce_ref[...] = ce.reshape(1, P)
    d = pred_ref[0] - gt_ref[0]
    ad = jnp.abs(d)
    sl1 = jnp.where(ad < 1.0, 0.5 * d * d, ad - 0.5)
    pos = lab2d > 0
    sl1_ref[0, 0] = jnp.sum(jnp.where(pos, sl1, 0.0))


def _stage_b_kernel(ukey_ref, ce_ref, lab_ref, sl1p_ref,
                    out_sl1_ref, out_cls_ref):
    lab = lab_ref[...]
    B, P = lab.shape
    pos = lab > 0
    valid = jnp.logical_not(pos)
    posf = pos.astype(jnp.float32)
    npos = jnp.sum(posf, axis=1, keepdims=True)
    k = jnp.minimum(_NEG_POS_RATIO * npos, jnp.float32(P) - npos)
    u = ukey_ref[...]
    T = jnp.zeros((B, 1), jnp.uint32)
    for bit in range(31, -1, -1):
        cand = T | jnp.uint32(2 ** bit)
        cnt = jnp.sum(jnp.where(valid & (u >= cand), 1.0, 0.0),
                      axis=1, keepdims=True)
        T = jnp.where(cnt >= k, cand, T)
    count_gt = jnp.sum(jnp.where(valid & (u > T), 1.0, 0.0),
                       axis=1, keepdims=True)
    need = k - count_gt
    eq = valid & (u == T)
    idx = lax.broadcasted_iota(jnp.int32, (B, P), 1)
    I = jnp.zeros((B, 1), jnp.int32)
    for bit in range(13, -1, -1):
        cand = I + (1 << bit)
        cnt = jnp.sum(jnp.where(eq & (idx < cand), 1.0, 0.0),
                      axis=1, keepdims=True)
        I = jnp.where(cnt <= need, cand, I)
    selected = (valid & (u > T)) | (eq & (idx < I))
    mask = pos | selected
    cls = jnp.sum(jnp.where(mask, ce_ref[...], 0.0))
    npos_tot = jnp.sum(posf)
    out_sl1_ref[0, 0] = jnp.sum(sl1p_ref[...]) / npos_tot
    out_cls_ref[0, 0] = cls / npos_tot


def kernel(confidence, predicted_locations, labels, gt_locations):
    B, P, C = confidence.shape
    predT = jnp.transpose(predicted_locations, (0, 2, 1))
    gtT = jnp.transpose(gt_locations, (0, 2, 1))

    ukey, ce, sl1p = pl.pallas_call(
        _stage_a_kernel,
        grid=(B,),
        in_specs=[
            pl.BlockSpec((1, P, C), lambda b: (b, 0, 0)),
            pl.BlockSpec((1, P), lambda b: (b, 0)),
            pl.BlockSpec((1, 4, P), lambda b: (b, 0, 0)),
            pl.BlockSpec((1, 4, P), lambda b: (b, 0, 0)),
        ],
        out_specs=[
            pl.BlockSpec((1, P), lambda b: (b, 0)),
            pl.BlockSpec((1, P), lambda b: (b, 0)),
            pl.BlockSpec((1, 1), lambda b: (b, 0)),
        ],
        out_shape=[
            jax.ShapeDtypeStruct((B, P), jnp.uint32),
            jax.ShapeDtypeStruct((B, P), jnp.float32),
            jax.ShapeDtypeStruct((B, 1), jnp.float32),
        ],
    )(confidence, labels, predT, gtT)

    out_sl1, out_cls = pl.pallas_call(
        _stage_b_kernel,
        out_shape=[
            jax.ShapeDtypeStruct((1, 1), jnp.float32),
            jax.ShapeDtypeStruct((1, 1), jnp.float32),
        ],
    )(ukey, ce, labels, sl1p)

    return (out_sl1[0, 0], out_cls[0, 0])


# speedup vs baseline: 1.7739x; 1.7739x over previous
import jax
import jax.numpy as jnp
from jax import lax
from jax.experimental import pallas as pl
from jax.experimental.pallas import tpu as pltpu

_NEG_POS_RATIO = 3


def _ordered_u32(x):
    b = lax.bitcast_convert_type(x, jnp.int32)
    key = b ^ ((b >> 31) | jnp.int32(-2147483648))
    return lax.bitcast_convert_type(key, jnp.uint32)


def _stage_a_kernel(conf_ref, lab_ref, pred_ref, gt_ref,
                    ukey_ref, ce_ref, sl1_ref):
    c = conf_ref[0]
    lab2d = lab_ref[0]
    P, C = c.shape
    m = jnp.max(c, axis=-1, keepdims=True)
    lse = jnp.log(jnp.sum(jnp.exp(c - m), axis=-1, keepdims=True)) + m
    bg = lse[:, 0:1] - c[:, 0:1]
    lab_col = lab2d.reshape(P, 1)
    cls_iota = lax.broadcasted_iota(jnp.int32, (P, C), 1)
    picked = jnp.sum(jnp.where(cls_iota == lab_col, c, 0.0),
                     axis=-1, keepdims=True)
    ce = lse - picked
    ukey_ref[0] = _ordered_u32(bg).reshape(1, P)
    ce_ref[0] = ce.reshape(1, P)
    d = pred_ref[0] - gt_ref[0]
    ad = jnp.abs(d)
    sl1 = jnp.where(ad < 1.0, 0.5 * d * d, ad - 0.5)
    pos = lab2d > 0
    sl1_ref[...] = jnp.sum(jnp.where(pos, sl1, 0.0)).reshape(1, 1, 1)


def _stage_b_kernel(ukey_ref, ce_ref, lab_ref, sl1p_ref,
                    out_sl1_ref, out_cls_ref):
    lab = lab_ref[...]
    B, P = lab.shape
    pos = lab > 0
    valid = jnp.logical_not(pos)
    posf = pos.astype(jnp.float32)
    npos = jnp.sum(posf, axis=1, keepdims=True)
    k = jnp.minimum(_NEG_POS_RATIO * npos, jnp.float32(P) - npos)
    u = ukey_ref[...]
    T = jnp.zeros((B, 1), jnp.uint32)
    for bit in range(31, -1, -1):
        cand = T | jnp.uint32(2 ** bit)
        cnt = jnp.sum(jnp.where(valid & (u >= cand), 1.0, 0.0),
                      axis=1, keepdims=True)
        T = jnp.where(cnt >= k, cand, T)
    count_gt = jnp.sum(jnp.where(valid & (u > T), 1.0, 0.0),
                       axis=1, keepdims=True)
    need = k - count_gt
    eq = valid & (u == T)
    idx = lax.broadcasted_iota(jnp.int32, (B, P), 1)
    I = jnp.zeros((B, 1), jnp.int32)
    for bit in range(13, -1, -1):
        cand = I + (1 << bit)
        cnt = jnp.sum(jnp.where(eq & (idx < cand), 1.0, 0.0),
                      axis=1, keepdims=True)
        I = jnp.where(cnt <= need, cand, I)
    selected = (valid & (u > T)) | (eq & (idx < I))
    mask = pos | selected
    cls = jnp.sum(jnp.where(mask, ce_ref[...], 0.0))
    npos_tot = jnp.sum(posf)
    out_sl1_ref[...] = (jnp.sum(sl1p_ref[...]) / npos_tot).reshape(1, 1)
    out_cls_ref[...] = (cls / npos_tot).reshape(1, 1)


def kernel(confidence, predicted_locations, labels, gt_locations):
    B, P, C = confidence.shape
    predT = jnp.transpose(predicted_locations, (0, 2, 1))
    gtT = jnp.transpose(gt_locations, (0, 2, 1))
    lab3 = labels.reshape(B, 1, P)

    ukey3, ce3, sl1p = pl.pallas_call(
        _stage_a_kernel,
        grid=(B,),
        in_specs=[
            pl.BlockSpec((1, P, C), lambda b: (b, 0, 0)),
            pl.BlockSpec((1, 1, P), lambda b: (b, 0, 0)),
            pl.BlockSpec((1, 4, P), lambda b: (b, 0, 0)),
            pl.BlockSpec((1, 4, P), lambda b: (b, 0, 0)),
        ],
        out_specs=[
            pl.BlockSpec((1, 1, P), lambda b: (b, 0, 0)),
            pl.BlockSpec((1, 1, P), lambda b: (b, 0, 0)),
            pl.BlockSpec((1, 1, 1), lambda b: (b, 0, 0)),
        ],
        out_shape=[
            jax.ShapeDtypeStruct((B, 1, P), jnp.uint32),
            jax.ShapeDtypeStruct((B, 1, P), jnp.float32),
            jax.ShapeDtypeStruct((B, 1, 1), jnp.float32),
        ],
    )(confidence, lab3, predT, gtT)
    ukey = ukey3.reshape(B, P)
    ce = ce3.reshape(B, P)
    sl1p = sl1p.reshape(B, 1)

    out_sl1, out_cls = pl.pallas_call(
        _stage_b_kernel,
        out_shape=[
            jax.ShapeDtypeStruct((1, 1), jnp.float32),
            jax.ShapeDtypeStruct((1, 1), jnp.float32),
        ],
    )(ukey, ce, labels, sl1p)

    return (out_sl1[0, 0], out_cls[0, 0])
